# baseline (speedup 1.0000x reference)
import functools

import jax
import jax.numpy as jnp
from jax.experimental import pallas as pl

MIN_WINDOW_SIZE = 33
MAX_WINDOW_SIZE = 99

_BR = 512


def _mask_kernel(boxes_ref, out_ref):
    r0 = pl.program_id(0) * _BR
    wh = boxes_ref[:, 2:4]
    mx = jnp.max(wh, axis=1)
    mn = jnp.min(wh, axis=1)
    scale = jnp.sqrt(mx / mn)
    window = (MIN_WINDOW_SIZE * scale).astype(jnp.int32)
    window = jnp.clip(window, MIN_WINDOW_SIZE, MAX_WINDOW_SIZE)
    half = window // 2

    n = out_ref.shape[1]
    i = r0 + jax.lax.broadcasted_iota(jnp.int32, (_BR, n), 0)
    j = jax.lax.broadcasted_iota(jnp.int32, (_BR, n), 1)
    h = half[:, None]
    out_ref[...] = (j >= i - h) & (j <= i + h)


@jax.jit
def kernel(boxes):
    n = boxes.shape[0]
    grid = (pl.cdiv(n, _BR),)
    return pl.pallas_call(
        _mask_kernel,
        grid=grid,
        in_specs=[pl.BlockSpec((_BR, 4), lambda r: (r, 0))],
        out_specs=pl.BlockSpec((_BR, n), lambda r: (r, 0)),
        out_shape=jax.ShapeDtypeStruct((n, n), jnp.bool_),
    )(boxes)
